# baseline (device time: 45258 ns/iter reference)
import functools

import jax
import jax.numpy as jnp
from jax import lax
from jax.experimental import pallas as pl
from jax.experimental.pallas import tpu as pltpu

N_DEV = 4
M, N = 2048, 1024
L = 2
CHUNK = M // N_DEV
CH = CHUNK // 2 // L
N_STEP = N_DEV - 1
N_TOT = 2 * N_STEP


def kernel(x):
    def body(x_ref, out_ref, xstage_ref, work_ref, recv_ref,
             fetch_sems, store_sems, send_sems, recv_sems):
        my = lax.axis_index("i")
        left = lax.rem(my + (N_DEV - 1), N_DEV)
        right = lax.rem(my + 1, N_DEV)

        barrier_sem = pltpu.get_barrier_semaphore()
        for nbr in (left, right):
            pl.semaphore_signal(
                barrier_sem, inc=1,
                device_id=(nbr,), device_id_type=pl.DeviceIdType.MESH,
            )

        chunks = [lax.rem(my + k, N_DEV) for k in range(N_DEV)]
        fetches = []
        for k, c in enumerate(chunks):
            f = pltpu.make_async_copy(
                x_ref.at[0, pl.ds(c * CHUNK, CHUNK), :],
                xstage_ref.at[k],
                fetch_sems.at[k],
            )
            f.start()
            fetches.append(f)

        def mk(s, l):
            if s < N_STEP:
                cw_send = lax.rem(my + (N_DEV - s), N_DEV)
                ccw_send = lax.rem(my + s, N_DEV)
                dst_cw = recv_ref.at[0, s, l]
                dst_ccw = recv_ref.at[1, s, l]
            else:
                t = s - N_STEP
                cw_send = lax.rem(my + (N_DEV + 1 - t), N_DEV)
                ccw_send = lax.rem(my + (N_DEV - 1 + t), N_DEV)
                dst_cw = work_ref.at[cw_send, 0, l]
                dst_ccw = work_ref.at[ccw_send, 1, l]
            r_cw = pltpu.make_async_remote_copy(
                src_ref=work_ref.at[cw_send, 0, l],
                dst_ref=dst_cw,
                send_sem=send_sems.at[0, s, l],
                recv_sem=recv_sems.at[0, s, l],
                device_id=(right,),
                device_id_type=pl.DeviceIdType.MESH,
            )
            r_ccw = pltpu.make_async_remote_copy(
                src_ref=work_ref.at[ccw_send, 1, l],
                dst_ref=dst_ccw,
                send_sem=send_sems.at[1, s, l],
                recv_sem=recv_sems.at[1, s, l],
                device_id=(left,),
                device_id_type=pl.DeviceIdType.MESH,
            )
            return r_cw, r_ccw

        fetches[0].wait()
        work_ref[chunks[0]] = (
            xstage_ref[0].astype(jnp.bfloat16).reshape(2, L, CH, N)
        )
        pl.semaphore_wait(barrier_sem, 2)
        rd = {}
        for l in range(L):
            rd[(0, l)] = mk(0, l)
            rd[(0, l)][0].start()
            rd[(0, l)][1].start()
        for k in (1, 2, 3):
            fetches[k].wait()
            work_ref[chunks[k]] = (
                xstage_ref[k].astype(jnp.bfloat16).reshape(2, L, CH, N)
            )

        def store_out(c, d, l, e):
            st = pltpu.make_async_copy(
                work_ref.at[c, d, l],
                out_ref.at[pl.ds(c * CHUNK + (d * L + l) * CH, CH), :],
                store_sems.at[d, e, l],
            )
            st.start()
            return st

        stores = []
        for s in range(N_TOT):
            for l in range(L):
                r_cw, r_ccw = rd[(s, l)]
                r_cw.wait_recv()
                r_ccw.wait_recv()
                if s < N_STEP:
                    cw_recv = lax.rem(my + (N_DEV - s - 1), N_DEV)
                    ccw_recv = lax.rem(my + s + 1, N_DEV)
                    work_ref[cw_recv, 0, l] = (
                        work_ref[cw_recv, 0, l] + recv_ref[0, s, l]
                    )
                    work_ref[ccw_recv, 1, l] = (
                        work_ref[ccw_recv, 1, l] + recv_ref[1, s, l]
                    )
                    if s == N_STEP - 1:
                        stores.append(store_out(cw_recv, 0, l, 0))
                        stores.append(store_out(ccw_recv, 1, l, 0))
                else:
                    t = s - N_STEP
                    stores.append(
                        store_out(lax.rem(my + (N_DEV - t), N_DEV), 0, l, t + 1)
                    )
                    stores.append(
                        store_out(lax.rem(my + t, N_DEV), 1, l, t + 1)
                    )
                if s + 1 < N_TOT:
                    rd[(s + 1, l)] = mk(s + 1, l)
                    rd[(s + 1, l)][0].start()
                    rd[(s + 1, l)][1].start()

        for s in range(N_TOT):
            for l in range(L):
                rd[(s, l)][0].wait_send()
                rd[(s, l)][1].wait_send()
        for st in stores:
            st.wait()

        @functools.partial(
            pl.run_scoped, second_barrier=pltpu.SemaphoreType.REGULAR
        )
        def _(second_barrier):
            for nbr in (left, right):
                pl.semaphore_signal(
                    second_barrier, inc=1,
                    device_id=(nbr,), device_id_type=pl.DeviceIdType.MESH,
                )
            pl.semaphore_wait(second_barrier, 2)

    return pl.pallas_call(
        body,
        out_shape=jax.ShapeDtypeStruct((M, N), jnp.bfloat16),
        in_specs=[pl.BlockSpec(memory_space=pl.ANY)],
        out_specs=pl.BlockSpec(memory_space=pl.ANY),
        scratch_shapes=[
            pltpu.VMEM((N_DEV, CHUNK, N), jnp.float32),
            pltpu.VMEM((N_DEV, 2, L, CH, N), jnp.bfloat16),
            pltpu.VMEM((2, N_STEP, L, CH, N), jnp.bfloat16),
            pltpu.SemaphoreType.DMA((N_DEV,)),
            pltpu.SemaphoreType.DMA((2, N_DEV, L)),
            pltpu.SemaphoreType.DMA((2, N_TOT, L)),
            pltpu.SemaphoreType.DMA((2, N_TOT, L)),
        ],
        compiler_params=pltpu.CompilerParams(collective_id=0),
    )(x)


# device time: 44552 ns/iter; 1.0158x vs baseline; 1.0158x over previous
import functools

import jax
import jax.numpy as jnp
from jax import lax
from jax.experimental import pallas as pl
from jax.experimental.pallas import tpu as pltpu

N_DEV = 4
M, N = 2048, 1024
L = 2
CHUNK = M // N_DEV
CH = CHUNK // 2 // L
N_STEP = N_DEV - 1
N_TOT = 2 * N_STEP


def kernel(x):
    def body(x_ref, out_ref, xstage_ref, work_ref, recv_ref,
             fetch_sems, store_sems, send_sems, recv_sems):
        my = lax.axis_index("i")
        left = lax.rem(my + (N_DEV - 1), N_DEV)
        right = lax.rem(my + 1, N_DEV)

        barrier_sem = pltpu.get_barrier_semaphore()
        for nbr in (left, right):
            pl.semaphore_signal(
                barrier_sem, inc=1,
                device_id=(nbr,), device_id_type=pl.DeviceIdType.MESH,
            )

        chunks = [lax.rem(my + k, N_DEV) for k in range(N_DEV)]
        fetches = []
        for k, c in enumerate(chunks):
            f = pltpu.make_async_copy(
                x_ref.at[0, pl.ds(c * CHUNK, CHUNK), :],
                xstage_ref.at[k],
                fetch_sems.at[k],
            )
            f.start()
            fetches.append(f)

        def mk(s, l):
            if s < N_STEP:
                cw_send = lax.rem(my + (N_DEV - s), N_DEV)
                ccw_send = lax.rem(my + s, N_DEV)
                dst_cw = recv_ref.at[0, s, l]
                dst_ccw = recv_ref.at[1, s, l]
            else:
                t = s - N_STEP
                cw_send = lax.rem(my + (N_DEV + 1 - t), N_DEV)
                ccw_send = lax.rem(my + (N_DEV - 1 + t), N_DEV)
                dst_cw = work_ref.at[cw_send, 0, l]
                dst_ccw = work_ref.at[ccw_send, 1, l]
            r_cw = pltpu.make_async_remote_copy(
                src_ref=work_ref.at[cw_send, 0, l],
                dst_ref=dst_cw,
                send_sem=send_sems.at[0, s, l],
                recv_sem=recv_sems.at[0, s, l],
                device_id=(right,),
                device_id_type=pl.DeviceIdType.MESH,
            )
            r_ccw = pltpu.make_async_remote_copy(
                src_ref=work_ref.at[ccw_send, 1, l],
                dst_ref=dst_ccw,
                send_sem=send_sems.at[1, s, l],
                recv_sem=recv_sems.at[1, s, l],
                device_id=(left,),
                device_id_type=pl.DeviceIdType.MESH,
            )
            return r_cw, r_ccw

        fetches[0].wait()
        work_ref[chunks[0]] = (
            xstage_ref[0].astype(jnp.bfloat16).reshape(2, L, CH, N)
        )
        pl.semaphore_wait(barrier_sem, 2)
        rd = {}
        for l in range(L):
            rd[(0, l)] = mk(0, l)
            rd[(0, l)][0].start()
            rd[(0, l)][1].start()
        for k in (1, 2, 3):
            fetches[k].wait()
            work_ref[chunks[k]] = (
                xstage_ref[k].astype(jnp.bfloat16).reshape(2, L, CH, N)
            )

        def store_out(c, d, l, e):
            st = pltpu.make_async_copy(
                work_ref.at[c, d, l],
                out_ref.at[pl.ds(c * CHUNK + (d * L + l) * CH, CH), :],
                store_sems.at[d, e, l],
            )
            st.start()
            return st

        stores = []
        for s in range(N_TOT):
            for l in range(L):
                r_cw, r_ccw = rd[(s, l)]
                if s + 1 < N_TOT:
                    rd[(s + 1, l)] = mk(s + 1, l)
                r_cw.wait_recv()
                if s < N_STEP:
                    cw_recv = lax.rem(my + (N_DEV - s - 1), N_DEV)
                    work_ref[cw_recv, 0, l] = (
                        work_ref[cw_recv, 0, l] + recv_ref[0, s, l]
                    )
                if s + 1 < N_TOT:
                    rd[(s + 1, l)][0].start()
                r_ccw.wait_recv()
                if s < N_STEP:
                    ccw_recv = lax.rem(my + s + 1, N_DEV)
                    work_ref[ccw_recv, 1, l] = (
                        work_ref[ccw_recv, 1, l] + recv_ref[1, s, l]
                    )
                if s + 1 < N_TOT:
                    rd[(s + 1, l)][1].start()
                if s == N_STEP - 1:
                    stores.append(store_out(cw_recv, 0, l, 0))
                    stores.append(store_out(ccw_recv, 1, l, 0))
                elif s >= N_STEP:
                    t = s - N_STEP
                    stores.append(
                        store_out(lax.rem(my + (N_DEV - t), N_DEV), 0, l, t + 1)
                    )
                    stores.append(
                        store_out(lax.rem(my + t, N_DEV), 1, l, t + 1)
                    )

        for s in range(N_TOT):
            for l in range(L):
                rd[(s, l)][0].wait_send()
                rd[(s, l)][1].wait_send()
        for st in stores:
            st.wait()


    return pl.pallas_call(
        body,
        out_shape=jax.ShapeDtypeStruct((M, N), jnp.bfloat16),
        in_specs=[pl.BlockSpec(memory_space=pl.ANY)],
        out_specs=pl.BlockSpec(memory_space=pl.ANY),
        scratch_shapes=[
            pltpu.VMEM((N_DEV, CHUNK, N), jnp.float32),
            pltpu.VMEM((N_DEV, 2, L, CH, N), jnp.bfloat16),
            pltpu.VMEM((2, N_STEP, L, CH, N), jnp.bfloat16),
            pltpu.SemaphoreType.DMA((N_DEV,)),
            pltpu.SemaphoreType.DMA((2, N_DEV, L)),
            pltpu.SemaphoreType.DMA((2, N_TOT, L)),
            pltpu.SemaphoreType.DMA((2, N_TOT, L)),
        ],
        compiler_params=pltpu.CompilerParams(collective_id=0),
    )(x)
